# Initial kernel scaffold; baseline (speedup 1.0000x reference)
import functools

import jax
import jax.numpy as jnp
from jax import lax
from jax.experimental import pallas as pl
from jax.experimental.pallas import tpu as pltpu
from jax.experimental.pallas import tpu_sc as plsc

_N = 10000
_E = 320000
_D_IN = 128
_D_HID = 128
_D_OUT = 64

_NC = 2
_NS = 16
_NW = _NC * _NS
_CH = 128
_NPAD = 10240
_RPT = _NPAD // _NS
_EPT = 10112
_NCHUNK = _EPT // _CH
_EPAD = _EPT * _NW - _E

_mesh = plsc.VectorSubcoreMesh(core_axis_name="c", subcore_axis_name="s")


def _make_agg(d):
    @functools.partial(
        pl.kernel,
        mesh=_mesh,
        out_type=jax.ShapeDtypeStruct((_NC, _NPAD, d), jnp.float32),
        scratch_types=[
            pltpu.VMEM_SHARED((_NPAD, d), jnp.float32),
            pltpu.VMEM((_CH,), jnp.int32),
            pltpu.VMEM((_CH,), jnp.int32),
            pltpu.VMEM((_CH, d), jnp.float32),
            pltpu.SemaphoreType.DMA,
        ],
    )
    def agg(h_hbm, src_hbm, dst_hbm, zeros_hbm, out_hbm,
            acc, src_idx, dst_idx, rows, sem):
        c = lax.axis_index("c")
        s = lax.axis_index("s")
        wid = c * _NS + s
        row0 = s * _RPT
        pltpu.sync_copy(zeros_hbm, acc.at[pl.ds(row0, _RPT)])
        plsc.subcore_barrier()
        base = wid * _EPT

        def chunk(j, carry):
            off = pl.multiple_of(base + j * _CH, _CH)
            pltpu.sync_copy(src_hbm.at[pl.ds(off, _CH)], src_idx)
            pltpu.sync_copy(dst_hbm.at[pl.ds(off, _CH)], dst_idx)
            pltpu.async_copy(h_hbm.at[src_idx], rows, sem).wait()
            pltpu.sync_copy(rows, acc.at[dst_idx], add=True)
            return carry

        lax.fori_loop(0, _NCHUNK, chunk, 0)
        plsc.subcore_barrier()
        pltpu.sync_copy(acc.at[pl.ds(row0, _RPT)],
                        out_hbm.at[c, pl.ds(row0, _RPT)])

    return agg


_agg_hid = _make_agg(_D_HID)
_agg_out = _make_agg(_D_OUT)


@functools.partial(
    pl.kernel,
    mesh=_mesh,
    out_type=jax.ShapeDtypeStruct((_NC, _NPAD), jnp.float32),
    scratch_types=[
        pltpu.VMEM_SHARED((_NPAD,), jnp.float32),
        pltpu.VMEM((_CH,), jnp.int32),
        pltpu.VMEM((_CH,), jnp.float32),
    ],
)
def _deg(dst_hbm, zeros1_hbm, ones_hbm, out_hbm, acc, dst_idx, ones_v):
    c = lax.axis_index("c")
    s = lax.axis_index("s")
    wid = c * _NS + s
    row0 = s * _RPT
    pltpu.sync_copy(zeros1_hbm, acc.at[pl.ds(row0, _RPT)])
    pltpu.sync_copy(ones_hbm, ones_v)
    plsc.subcore_barrier()
    base = wid * _EPT

    def chunk(j, carry):
        off = pl.multiple_of(base + j * _CH, _CH)
        pltpu.sync_copy(dst_hbm.at[pl.ds(off, _CH)], dst_idx)
        pltpu.sync_copy(ones_v, acc.at[dst_idx], add=True)
        return carry

    lax.fori_loop(0, _NCHUNK, chunk, 0)
    plsc.subcore_barrier()
    pltpu.sync_copy(acc.at[pl.ds(row0, _RPT)],
                    out_hbm.at[c, pl.ds(row0, _RPT)])


def _tc_pre_body(degp_ref, x_ref, w_ref, ht_ref, dinv_ref):
    deg = degp_ref[0] + degp_ref[1] + 1.0
    dinv = lax.rsqrt(deg)
    dinv_ref[...] = dinv
    h = jnp.dot(x_ref[...], w_ref[...], preferred_element_type=jnp.float32)
    ht_ref[...] = h * dinv


_tc_pre = pl.pallas_call(
    _tc_pre_body,
    out_shape=(
        jax.ShapeDtypeStruct((_N, _D_HID), jnp.float32),
        jax.ShapeDtypeStruct((_N, 1), jnp.float32),
    ),
)


def _make_tc_mid(d_out):
    def body(yp_ref, ht_ref, dinv_ref, b_ref, g_ref, bt_ref, wn_ref, out_ref):
        y = yp_ref[0, : _N, :] + yp_ref[1, : _N, :] + ht_ref[...]
        z = y * dinv_ref[...] + b_ref[...]
        mu = jnp.mean(z, axis=0, keepdims=True)
        var = jnp.mean(z * z, axis=0, keepdims=True) - mu * mu
        f = (z - mu) * lax.rsqrt(var + 1e-5) * g_ref[...] + bt_ref[...]
        f = jnp.maximum(f, 0.0)
        h = jnp.dot(f, wn_ref[...], preferred_element_type=jnp.float32)
        out_ref[...] = h * dinv_ref[...]

    return pl.pallas_call(
        body,
        out_shape=jax.ShapeDtypeStruct((_N, d_out), jnp.float32),
    )


_tc_mid_hid = _make_tc_mid(_D_HID)
_tc_mid_out = _make_tc_mid(_D_OUT)


def _tc_post_body(yp_ref, ht_ref, dinv_ref, b_ref, out_ref):
    y = yp_ref[0, : _N, :] + yp_ref[1, : _N, :] + ht_ref[...]
    z = y * dinv_ref[...] + b_ref[...]
    m = jnp.max(z, axis=1, keepdims=True)
    lse = jnp.log(jnp.sum(jnp.exp(z - m), axis=1, keepdims=True)) + m
    out_ref[...] = z - lse


_tc_post = pl.pallas_call(
    _tc_post_body,
    out_shape=jax.ShapeDtypeStruct((_N, _D_OUT), jnp.float32),
)


def kernel(x, edge_index, W0, b0, g0, bt0, W1, b1, g1, bt1, W2, b2):
    src = jnp.concatenate(
        [edge_index[0], jnp.zeros((_EPAD,), jnp.int32)])
    dst = jnp.concatenate(
        [edge_index[1], jnp.full((_EPAD,), _N, jnp.int32)])
    zeros_hid = jnp.zeros((_RPT, _D_HID), jnp.float32)
    zeros_out = jnp.zeros((_RPT, _D_OUT), jnp.float32)
    zeros1 = jnp.zeros((_RPT,), jnp.float32)
    ones = jnp.ones((_CH,), jnp.float32)

    degp = _deg(dst, zeros1, ones)
    degp_col = degp[:, : _N, None]

    ht0, dinv = _tc_pre(degp_col, x, W0)
    yp0 = _agg_hid(ht0, src, dst, zeros_hid)
    ht1 = _tc_mid_hid(yp0, ht0, dinv, b0[None, :], g0[None, :],
                      bt0[None, :], W1)
    yp1 = _agg_hid(ht1, src, dst, zeros_hid)
    ht2 = _tc_mid_out(yp1, ht1, dinv, b1[None, :], g1[None, :],
                      bt1[None, :], W2)
    yp2 = _agg_out(ht2, src, dst, zeros_out)
    return _tc_post(yp2, ht2, dinv, b2[None, :])


# speedup vs baseline: 9.2940x; 9.2940x over previous
import functools

import jax
import jax.numpy as jnp
from jax import lax
from jax.experimental import pallas as pl
from jax.experimental.pallas import tpu as pltpu
from jax.experimental.pallas import tpu_sc as plsc

_N = 10000
_E = 320000
_D_IN = 128
_D_HID = 128
_D_OUT = 64

_NC = 2
_NS = 16
_NW = _NC * _NS
_CH = 128
_NPAD = 10240
_RPT = _NPAD // _NS
_EPT = 10112
_NCHUNK = _EPT // _CH
_EPAD = _EPT * _NW - _E

_mesh = plsc.VectorSubcoreMesh(core_axis_name="c", subcore_axis_name="s")


def _make_agg(d):
    @functools.partial(
        pl.kernel,
        mesh=_mesh,
        out_type=jax.ShapeDtypeStruct((_NC, _NPAD, d), jnp.float32),
        scratch_types=[
            pltpu.VMEM_SHARED((_NPAD, d), jnp.float32),
            pltpu.VMEM((_CH,), jnp.int32),
            pltpu.VMEM((_CH,), jnp.int32),
            pltpu.VMEM((_CH, d), jnp.float32),
            pltpu.SemaphoreType.DMA,
        ],
    )
    def agg(h_hbm, src_hbm, dst_hbm, zeros_hbm, out_hbm,
            acc, src_idx, dst_idx, rows, sem):
        c = lax.axis_index("c")
        s = lax.axis_index("s")
        wid = c * _NS + s
        row0 = s * _RPT
        pltpu.sync_copy(zeros_hbm, acc.at[pl.ds(row0, _RPT)])
        plsc.subcore_barrier()
        base = wid * _EPT

        def chunk(j, carry):
            off = pl.multiple_of(base + j * _CH, _CH)
            pltpu.sync_copy(src_hbm.at[pl.ds(off, _CH)], src_idx)
            pltpu.sync_copy(dst_hbm.at[pl.ds(off, _CH)], dst_idx)
            pltpu.async_copy(h_hbm.at[src_idx], rows, sem).wait()
            pltpu.sync_copy(rows, acc.at[dst_idx], add=True)
            return carry

        lax.fori_loop(0, _NCHUNK, chunk, 0)
        plsc.subcore_barrier()
        pltpu.sync_copy(acc.at[pl.ds(row0, _RPT)],
                        out_hbm.at[c, pl.ds(row0, _RPT)])

    return agg


_agg_hid = _make_agg(_D_HID)


@functools.partial(
    pl.kernel,
    mesh=_mesh,
    out_type=jax.ShapeDtypeStruct((_NC, _NPAD), jnp.float32),
    scratch_types=[
        pltpu.VMEM_SHARED((_NPAD,), jnp.float32),
        pltpu.VMEM((_CH,), jnp.int32),
        pltpu.VMEM((_CH,), jnp.float32),
    ],
)
def _deg(dst_hbm, zeros1_hbm, ones_hbm, out_hbm, acc, dst_idx, ones_v):
    c = lax.axis_index("c")
    s = lax.axis_index("s")
    wid = c * _NS + s
    row0 = s * _RPT
    pltpu.sync_copy(zeros1_hbm, acc.at[pl.ds(row0, _RPT)])
    pltpu.sync_copy(ones_hbm, ones_v)
    plsc.subcore_barrier()
    base = wid * _EPT

    def chunk(j, carry):
        off = pl.multiple_of(base + j * _CH, _CH)
        pltpu.sync_copy(dst_hbm.at[pl.ds(off, _CH)], dst_idx)
        pltpu.sync_copy(ones_v, acc.at[dst_idx], add=True)
        return carry

    lax.fori_loop(0, _NCHUNK, chunk, 0)
    plsc.subcore_barrier()
    pltpu.sync_copy(acc.at[pl.ds(row0, _RPT)],
                    out_hbm.at[c, pl.ds(row0, _RPT)])


def _tc_pre_body(degp_ref, x_ref, w_ref, ht_ref, dinv_ref):
    deg = degp_ref[0] + degp_ref[1] + 1.0
    dinv = lax.rsqrt(deg)
    dinv_ref[...] = dinv
    h = jnp.dot(x_ref[...], w_ref[...], preferred_element_type=jnp.float32)
    ht_ref[...] = h * dinv


_tc_pre = pl.pallas_call(
    _tc_pre_body,
    out_shape=(
        jax.ShapeDtypeStruct((_N, _D_HID), jnp.float32),
        jax.ShapeDtypeStruct((_N, 1), jnp.float32),
    ),
)


def _make_tc_mid(d_out):
    def body(yp_ref, ht_ref, dinv_ref, b_ref, g_ref, bt_ref, wn_ref, out_ref):
        y = yp_ref[0, : _N, :] + yp_ref[1, : _N, :] + ht_ref[...]
        z = y * dinv_ref[...] + b_ref[...]
        mu = jnp.mean(z, axis=0, keepdims=True)
        var = jnp.mean(z * z, axis=0, keepdims=True) - mu * mu
        f = (z - mu) * lax.rsqrt(var + 1e-5) * g_ref[...] + bt_ref[...]
        f = jnp.maximum(f, 0.0)
        h = jnp.dot(f, wn_ref[...], preferred_element_type=jnp.float32)
        if d_out == _D_HID:
            out_ref[...] = h * dinv_ref[...]
        else:
            out_ref[:, :d_out] = h * dinv_ref[...]
            out_ref[:, d_out:] = jnp.zeros((_N, _D_HID - d_out), jnp.float32)

    return pl.pallas_call(
        body,
        out_shape=jax.ShapeDtypeStruct((_N, _D_HID), jnp.float32),
    )


_tc_mid_hid = _make_tc_mid(_D_HID)
_tc_mid_out = _make_tc_mid(_D_OUT)


def _tc_post_body(yp_ref, ht_ref, dinv_ref, b_ref, out_ref):
    y = (yp_ref[0, : _N, : _D_OUT] + yp_ref[1, : _N, : _D_OUT]
         + ht_ref[:, : _D_OUT])
    z = y * dinv_ref[...] + b_ref[...]
    m = jnp.max(z, axis=1, keepdims=True)
    lse = jnp.log(jnp.sum(jnp.exp(z - m), axis=1, keepdims=True)) + m
    out_ref[...] = z - lse


_tc_post = pl.pallas_call(
    _tc_post_body,
    out_shape=jax.ShapeDtypeStruct((_N, _D_OUT), jnp.float32),
)


def kernel(x, edge_index, W0, b0, g0, bt0, W1, b1, g1, bt1, W2, b2):
    src = jnp.concatenate(
        [edge_index[0], jnp.zeros((_EPAD,), jnp.int32)])
    dst = jnp.concatenate(
        [edge_index[1], jnp.full((_EPAD,), _N, jnp.int32)])
    zeros_hid = jnp.zeros((_RPT, _D_HID), jnp.float32)
    zeros1 = jnp.zeros((_RPT,), jnp.float32)
    ones = jnp.ones((_CH,), jnp.float32)

    degp = _deg(dst, zeros1, ones)
    degp_col = degp[:, : _N, None]

    ht0, dinv = _tc_pre(degp_col, x, W0)
    yp0 = _agg_hid(ht0, src, dst, zeros_hid)
    ht1 = _tc_mid_hid(yp0, ht0, dinv, b0[None, :], g0[None, :],
                      bt0[None, :], W1)
    yp1 = _agg_hid(ht1, src, dst, zeros_hid)
    ht2 = _tc_mid_out(yp1, ht1, dinv, b1[None, :], g1[None, :],
                      bt1[None, :], W2)
    yp2 = _agg_hid(ht2, src, dst, zeros_hid)
    return _tc_post(yp2, ht2, dinv, b2[None, :])
